# baseline (speedup 1.0000x reference)
import jax
import jax.numpy as jnp
from jax import lax
from jax.experimental import pallas as pl
from jax.experimental.pallas import tpu as pltpu
from jax.experimental.pallas import tpu_sc as plsc

H = 24
W = 24
HW = H * W
NPLANES = 64 * 384
NTILES = 32
PLANES_PER_TILE = NPLANES // NTILES
CHUNK = 64
NCHUNKS = PLANES_PER_TILE // CHUNK
NGROUPS = CHUNK // 16
NV = HW // 16
HALF = 3


def _tile_body(x_hbm, t_hbm, o_hbm, tbuf, buf, sem):
    del sem
    wid = lax.axis_index("s") * 2 + lax.axis_index("c")
    tile_base = wid * PLANES_PER_TILE
    pltpu.sync_copy(t_hbm.at[pl.ds(tile_base, PLANES_PER_TILE)], tbuf)

    lane = lax.iota(jnp.int32, 16)
    krs, kcs = [], []
    for v in range(3):
        k = lane + v * 16
        kr = k // 6
        krs.append(kr)
        kcs.append(k - kr * 6)
    zero16 = jnp.zeros((16,), jnp.float32)
    neginf = jnp.full((16,), -3.4e38, jnp.float32)

    def group_body(ci, g):
        base0 = g * (16 * HW)
        idx0 = base0 + lane * HW

        def step(j, carry):
            m, jb = carry
            v = plsc.load_gather(buf, [idx0 + j])
            gt = v > m
            return jnp.where(gt, v, m), jnp.where(gt, j, jb)

        _, jb = lax.fori_loop(0, HW, step,
                              (neginf, jnp.zeros((16,), jnp.int32)),
                              unroll=8)

        hh = jb // W
        ww = jb - hh * W
        h1 = jnp.clip(hh - HALF, 0, H - 1)
        h2 = jnp.clip(hh + HALF, 0, H - 1)
        w1 = jnp.clip(ww - HALF, 0, W - 1)
        w2 = jnp.clip(ww + HALF, 0, W - 1)
        dh = h2 - h1
        dw = w2 - w1
        area = (dh * dw).astype(jnp.float32)
        lam = jnp.float32(HW) / (jnp.float32(HW) - area)

        tv = tbuf[pl.ds(ci * CHUNK + g * 16, 16)]

        for p in range(16):
            tp = tv[p]

            @pl.when(tp != 0.0)
            def _(p=p):
                lamp = lam[p]
                pb = base0 + p * HW
                for c in range(NV):
                    sl = pl.ds(pb + c * 16, 16)
                    buf[sl] = buf[sl] * lamp
                h1p = h1[p]
                w1p = w1[p]
                dhp = dh[p]
                dwp = dw[p]
                for v in range(3):
                    bidx = pb + (h1p + krs[v]) * W + (w1p + kcs[v])
                    msk = (krs[v] < dhp) & (kcs[v] < dwp)
                    plsc.store_scatter(buf, [bidx], zero16, mask=msk)

    def chunk_body(ci, carry):
        base_el = (tile_base + ci * CHUNK) * HW
        pltpu.sync_copy(x_hbm.at[pl.ds(base_el, CHUNK * HW)], buf)
        lax.fori_loop(0, NGROUPS, lambda g, c: (group_body(ci, g), c)[1], 0,
                      unroll=False)
        pltpu.sync_copy(buf, o_hbm.at[pl.ds(base_el, CHUNK * HW)])
        return carry

    lax.fori_loop(0, NCHUNKS, chunk_body, 0, unroll=False)


@jax.jit
def kernel(x, T):
    batch, channel, h, w = x.shape
    xf = x.reshape(-1)
    tf = (T.reshape(-1) != 0.0).astype(jnp.float32)
    mesh = plsc.VectorSubcoreMesh(core_axis_name="c", subcore_axis_name="s")
    run = pl.kernel(
        _tile_body,
        out_type=jax.ShapeDtypeStruct((NPLANES * HW,), jnp.float32),
        mesh=mesh,
        scratch_types=[
            pltpu.VMEM((PLANES_PER_TILE,), jnp.float32),
            pltpu.VMEM((CHUNK * HW,), jnp.float32),
            pltpu.SemaphoreType.DMA,
        ],
        compiler_params=pltpu.CompilerParams(needs_layout_passes=False),
    )
    out = run(xf, tf)
    return out.reshape(batch, channel, h, w)


# speedup vs baseline: 1.0173x; 1.0173x over previous
import jax
import jax.numpy as jnp
from jax import lax
from jax.experimental import pallas as pl
from jax.experimental.pallas import tpu as pltpu
from jax.experimental.pallas import tpu_sc as plsc

H = 24
W = 24
HW = H * W
NPLANES = 64 * 384
NTILES = 32
PLANES_PER_TILE = NPLANES // NTILES
CHUNK = 64
NCHUNKS = PLANES_PER_TILE // CHUNK
NGROUPS = CHUNK // 16
NV = HW // 16
HALF = 3


def _tile_body(x_hbm, t_hbm, o_hbm, tbuf, buf0, buf1, is0, is1, os0, os1):
    wid = lax.axis_index("s") * 2 + lax.axis_index("c")
    tile_base = wid * PLANES_PER_TILE
    pltpu.sync_copy(t_hbm.at[pl.ds(tile_base, PLANES_PER_TILE)], tbuf)

    bufs = (buf0, buf1)
    isems = (is0, is1)
    osems = (os0, os1)

    lane = lax.iota(jnp.int32, 16)
    krs, kcs = [], []
    for v in range(3):
        k = lane + v * 16
        kr = k // 6
        krs.append(kr)
        kcs.append(k - kr * 6)
    zero16 = jnp.zeros((16,), jnp.float32)
    neginf = jnp.full((16,), -3.4e38, jnp.float32)

    def in_copy(ci, b):
        base_el = (tile_base + ci * CHUNK) * HW
        return pltpu.make_async_copy(
            x_hbm.at[pl.ds(base_el, CHUNK * HW)], bufs[b], isems[b])

    def out_copy(ci, b):
        base_el = (tile_base + ci * CHUNK) * HW
        return pltpu.make_async_copy(
            bufs[b], o_hbm.at[pl.ds(base_el, CHUNK * HW)], osems[b])

    def group_body(buf, ci, g):
        base0 = g * (16 * HW)
        idx0 = base0 + lane * HW

        def step(j, carry):
            m, jb = carry
            v = plsc.load_gather(buf, [idx0 + j])
            gt = v > m
            return jnp.where(gt, v, m), jnp.where(gt, j, jb)

        _, jb = lax.fori_loop(0, HW, step,
                              (neginf, jnp.zeros((16,), jnp.int32)),
                              unroll=8)

        hh = jb // W
        ww = jb - hh * W
        h1 = jnp.clip(hh - HALF, 0, H - 1)
        h2 = jnp.clip(hh + HALF, 0, H - 1)
        w1 = jnp.clip(ww - HALF, 0, W - 1)
        w2 = jnp.clip(ww + HALF, 0, W - 1)
        dh = h2 - h1
        dw = w2 - w1
        area = (dh * dw).astype(jnp.float32)
        lam = jnp.float32(HW) / (jnp.float32(HW) - area)

        tv = tbuf[pl.ds(ci * CHUNK + g * 16, 16)]

        for p in range(16):
            tp = tv[p]

            @pl.when(tp != 0.0)
            def _(p=p):
                lamp = lam[p]
                pb = base0 + p * HW
                for c in range(NV):
                    sl = pl.ds(pb + c * 16, 16)
                    buf[sl] = buf[sl] * lamp
                h1p = h1[p]
                w1p = w1[p]
                dhp = dh[p]
                dwp = dw[p]
                for v in range(3):
                    bidx = pb + (h1p + krs[v]) * W + (w1p + kcs[v])
                    msk = (krs[v] < dhp) & (kcs[v] < dwp)
                    plsc.store_scatter(buf, [bidx], zero16, mask=msk)

    in_copy(0, 0).start()
    in_copy(1, 1).start()

    def superstep(s, carry):
        for b in range(2):
            ci = 2 * s + b
            in_copy(ci, b).wait()
            lax.fori_loop(0, NGROUPS,
                          lambda g, c, b=b, ci=ci:
                          (group_body(bufs[b], ci, g), c)[1],
                          0, unroll=False)
            out_copy(ci, b).start()

            @pl.when(ci + 2 < NCHUNKS)
            def _(b=b, ci=ci):
                out_copy(ci, b).wait()
                in_copy(ci + 2, b).start()

        return carry

    lax.fori_loop(0, NCHUNKS // 2, superstep, 0, unroll=False)
    out_copy(NCHUNKS - 2, 0).wait()
    out_copy(NCHUNKS - 1, 1).wait()


@jax.jit
def kernel(x, T):
    batch, channel, h, w = x.shape
    xf = x.reshape(-1)
    tf = (T.reshape(-1) != 0.0).astype(jnp.float32)
    mesh = plsc.VectorSubcoreMesh(core_axis_name="c", subcore_axis_name="s")
    run = pl.kernel(
        _tile_body,
        out_type=jax.ShapeDtypeStruct((NPLANES * HW,), jnp.float32),
        mesh=mesh,
        scratch_types=[
            pltpu.VMEM((PLANES_PER_TILE,), jnp.float32),
            pltpu.VMEM((CHUNK * HW,), jnp.float32),
            pltpu.VMEM((CHUNK * HW,), jnp.float32),
            pltpu.SemaphoreType.DMA,
            pltpu.SemaphoreType.DMA,
            pltpu.SemaphoreType.DMA,
            pltpu.SemaphoreType.DMA,
        ],
        compiler_params=pltpu.CompilerParams(needs_layout_passes=False),
    )
    out = run(xf, tf)
    return out.reshape(batch, channel, h, w)


# speedup vs baseline: 9.3780x; 9.2181x over previous
import jax
import jax.numpy as jnp
from jax import lax
from jax.experimental import pallas as pl
from jax.experimental.pallas import tpu as pltpu
from jax.experimental.pallas import tpu_sc as plsc

H = 24
W = 24
HW = H * W
B = 64
C = 384
NROWS = B * HW
NCG = C // 16
NTILES = 32
CROWS = 96
NCHUNKS = HW // CROWS
HALF = 3
BMAX = 2 * HALF


def _tile_body(x_hbm, t_hbm, o_hbm, tbuf, buf0, buf1,
               macc, jacc, lamb, boxb, dhb, dwb, is0, is1, os0, os1):
    wid = lax.axis_index("s") * 2 + lax.axis_index("c")
    pltpu.sync_copy(t_hbm.at[pl.ds(wid * (2 * C), 2 * C)], tbuf)

    bufs = (buf0, buf1)
    isems = (is0, is1)
    osems = (os0, os1)
    lane = lax.iota(jnp.int32, 16)
    zero16 = jnp.zeros((16,), jnp.float32)
    neginf = jnp.full((16,), -3.4e38, jnp.float32)
    zeroi = jnp.zeros((16,), jnp.int32)

    def in_copy(rbase, q, bb):
        return pltpu.make_async_copy(
            x_hbm.at[pl.ds(rbase + q * CROWS, CROWS)], bufs[bb], isems[bb])

    def out_copy(rbase, q, bb):
        return pltpu.make_async_copy(
            bufs[bb], o_hbm.at[pl.ds(rbase + q * CROWS, CROWS)], osems[bb])

    for bb in range(2):
        rbase = (2 * wid + bb) * HW

        def init_cg(cg, carry):
            macc[cg, :] = neginf
            jacc[cg, :] = zeroi
            return carry

        lax.fori_loop(0, NCG, init_cg, 0, unroll=False)

        def p1_cg(buf, q, cg):
            sl = pl.ds(cg * 16, 16)
            e0 = q * CROWS

            def step(e, carry):
                m, jb = carry
                v = buf[e, sl]
                gt = v > m
                return jnp.maximum(m, v), jnp.where(gt, e0 + e, jb)

            m, jb = lax.fori_loop(0, CROWS, step,
                                  (macc[cg, :], jacc[cg, :]), unroll=8)
            macc[cg, :] = m
            jacc[cg, :] = jb

        in_copy(rbase, 0, 0).start()
        in_copy(rbase, 1, 1).start()

        def p1_super(s, carry):
            for b01 in range(2):
                q = 2 * s + b01
                in_copy(rbase, q, b01).wait()
                lax.fori_loop(0, NCG,
                              lambda cg, c, q=q, b01=b01:
                              (p1_cg(bufs[b01], q, cg), c)[1],
                              0, unroll=False)

                @pl.when(q + 2 < NCHUNKS)
                def _(q=q, b01=b01):
                    in_copy(rbase, q + 2, b01).start()

            return carry

        lax.fori_loop(0, NCHUNKS // 2, p1_super, 0, unroll=False)

        def params_cg(cg, carry):
            jb = jacc[cg, :]
            hh = jb // W
            ww = jb - hh * W
            h1 = jnp.clip(hh - HALF, 0, H - 1)
            h2 = jnp.clip(hh + HALF, 0, H - 1)
            w1 = jnp.clip(ww - HALF, 0, W - 1)
            w2 = jnp.clip(ww + HALF, 0, W - 1)
            dh = h2 - h1
            dw = w2 - w1
            area = (dh * dw).astype(jnp.float32)
            lam = jnp.float32(HW) / (jnp.float32(HW) - area)
            tv = tbuf[pl.ds(bb * C + cg * 16, 16)]
            active = tv != 0.0
            lamb[cg, :] = jnp.where(active, lam, 1.0)
            boxb[cg, :] = h1 * W + w1
            dhb[cg, :] = jnp.where(active, dh, 0)
            dwb[cg, :] = dw
            return carry

        lax.fori_loop(0, NCG, params_cg, 0, unroll=False)

        def p2_cg(buf, q, cg):
            sl = pl.ds(cg * 16, 16)
            lam = lamb[cg, :]
            base = boxb[cg, :]
            dh = dhb[cg, :]
            dw = dwb[cg, :]
            col = cg * 16 + lane
            lo = q * CROWS

            def step(e, carry):
                buf[e, sl] = buf[e, sl] * lam
                return carry

            lax.fori_loop(0, CROWS, step, 0, unroll=8)

            rel = base - lo
            for i in range(BMAX):
                for j in range(BMAX):
                    rloc = rel + (i * W + j)
                    msk = ((i < dh) & (j < dw)
                           & (rloc >= 0) & (rloc < CROWS))
                    plsc.store_scatter(buf, [rloc, col], zero16, mask=msk)

        in_copy(rbase, 0, 0).start()
        in_copy(rbase, 1, 1).start()

        def p2_super(s, carry):
            for b01 in range(2):
                q = 2 * s + b01
                in_copy(rbase, q, b01).wait()
                lax.fori_loop(0, NCG,
                              lambda cg, c, q=q, b01=b01:
                              (p2_cg(bufs[b01], q, cg), c)[1],
                              0, unroll=False)
                out_copy(rbase, q, b01).start()

                @pl.when(q + 2 < NCHUNKS)
                def _(q=q, b01=b01):
                    out_copy(rbase, q, b01).wait()
                    in_copy(rbase, q + 2, b01).start()

            return carry

        lax.fori_loop(0, NCHUNKS // 2, p2_super, 0, unroll=False)
        out_copy(rbase, NCHUNKS - 2, 0).wait()
        out_copy(rbase, NCHUNKS - 1, 1).wait()


@jax.jit
def kernel(x, T):
    batch, channel, h, w = x.shape
    x2 = jnp.transpose(x, (0, 2, 3, 1)).reshape(NROWS, C)
    tf = (T.reshape(-1) != 0.0).astype(jnp.float32)
    mesh = plsc.VectorSubcoreMesh(core_axis_name="c", subcore_axis_name="s")
    run = pl.kernel(
        _tile_body,
        out_type=jax.ShapeDtypeStruct((NROWS, C), jnp.float32),
        mesh=mesh,
        scratch_types=[
            pltpu.VMEM((2 * C,), jnp.float32),
            pltpu.VMEM((CROWS, C), jnp.float32),
            pltpu.VMEM((CROWS, C), jnp.float32),
            pltpu.VMEM((NCG, 16), jnp.float32),
            pltpu.VMEM((NCG, 16), jnp.int32),
            pltpu.VMEM((NCG, 16), jnp.float32),
            pltpu.VMEM((NCG, 16), jnp.int32),
            pltpu.VMEM((NCG, 16), jnp.int32),
            pltpu.VMEM((NCG, 16), jnp.int32),
            pltpu.SemaphoreType.DMA,
            pltpu.SemaphoreType.DMA,
            pltpu.SemaphoreType.DMA,
            pltpu.SemaphoreType.DMA,
        ],
        compiler_params=pltpu.CompilerParams(needs_layout_passes=False),
    )
    out = run(x2, tf)
    return jnp.transpose(out.reshape(batch, h, w, channel), (0, 3, 1, 2))
